# initial kernel scaffold (unmeasured)
import jax
import jax.numpy as jnp
from jax import lax
from jax.experimental import pallas as pl
from jax.experimental.pallas import tpu as pltpu


def kernel(dy, W):
    dyb = dy.astype(jnp.bfloat16)
    Wb = W.astype(jnp.bfloat16)
    partial = lax.dot_general(
        dyb, Wb,
        dimension_numbers=(((1,), (1,)), ((), ())),
        preferred_element_type=jnp.float32,
    )
    pb = partial.astype(jnp.bfloat16)

    m, n = partial.shape

    def body(pf32_ref, pb_ref, out_ref, recv_buf, send_sem, recv_sem):
        my_x = lax.axis_index("x")
        my_y = lax.axis_index("y")
        my_z = lax.axis_index("z")
        nbr = (my_x, 1 - my_y, my_z)

        barrier = pltpu.get_barrier_semaphore()
        pl.semaphore_signal(
            barrier, inc=1, device_id=nbr,
            device_id_type=pl.DeviceIdType.MESH,
        )
        pl.semaphore_wait(barrier, 1)

        rdma = pltpu.make_async_remote_copy(
            src_ref=pb_ref,
            dst_ref=recv_buf,
            send_sem=send_sem,
            recv_sem=recv_sem,
            device_id=nbr,
            device_id_type=pl.DeviceIdType.MESH,
        )
        rdma.start()
        rdma.wait()

        out_ref[...] = pf32_ref[...] + recv_buf[...].astype(jnp.float32)

    return pl.pallas_call(
        body,
        out_shape=jax.ShapeDtypeStruct((m, n), jnp.float32),
        in_specs=[
            pl.BlockSpec(memory_space=pltpu.VMEM),
            pl.BlockSpec(memory_space=pltpu.VMEM),
        ],
        out_specs=pl.BlockSpec(memory_space=pltpu.VMEM),
        scratch_shapes=[
            pltpu.VMEM((m, n), jnp.bfloat16),
            pltpu.SemaphoreType.DMA,
            pltpu.SemaphoreType.DMA,
        ],
        compiler_params=pltpu.CompilerParams(collective_id=0),
    )(partial, pb)


# baseline (device time: 97162 ns/iter reference)
import jax
import jax.numpy as jnp
from jax import lax
from jax.experimental import pallas as pl
from jax.experimental.pallas import tpu as pltpu

M = 2048
K = 8192
N = 2048
MZ = M // 2
NC = 8
CN = N // NC
NR = 4
CR = MZ // NR

BF = jnp.bfloat16
F32 = jnp.float32


def kernel(dy, W):
    def body(dy_hbm, w_hbm, out_hbm,
             stage, dy_bf, w_bf, p_buf, ysend, yrecv, zsend, zrecv, zf32,
             stage_sems, ysend_sems, yrecv_sems, zsend_sems, zrecv_sems,
             oy_sems, oz_sems):
        my_x = lax.axis_index("x")
        my_y = lax.axis_index("y")
        my_z = lax.axis_index("z")
        y_nbr = (my_x, 1 - my_y, my_z)
        z_nbr = (my_x, my_y, 1 - my_z)
        row0 = my_z * MZ

        def slot2(i):
            return lax.rem(i, 2) if not isinstance(i, int) else i % 2

        def dy_dma(i):
            s = slot2(i)
            return pltpu.make_async_copy(
                dy_hbm.at[pl.ds(row0 + i * CR, CR), :],
                stage.at[pl.ds(s * CN, CR), :],
                stage_sems.at[s],
            )

        def w_dma(c):
            s = slot2(c)
            return pltpu.make_async_copy(
                w_hbm.at[pl.ds(c * CN, CN), :],
                stage.at[pl.ds(s * CN, CN), :],
                stage_sems.at[s],
            )

        def oy_dma(k):
            s = slot2(k)
            return pltpu.make_async_copy(
                p_buf.at[pl.ds(s * MZ, MZ), :],
                out_hbm.at[pl.ds(row0, MZ), pl.ds(k * CN, CN)],
                oy_sems.at[s],
            )

        def oz_dma(k):
            s = slot2(k)
            return pltpu.make_async_copy(
                zf32.at[pl.ds(s * MZ, MZ), :],
                out_hbm.at[pl.ds((1 - my_z) * MZ, MZ), pl.ds(k * CN, CN)],
                oz_sems.at[s],
            )

        def y_rdma(c):
            s = slot2(c)
            return pltpu.make_async_remote_copy(
                src_ref=ysend.at[pl.ds(s * MZ, MZ), :],
                dst_ref=yrecv.at[pl.ds(c * MZ, MZ), :],
                send_sem=ysend_sems.at[s],
                recv_sem=yrecv_sems.at[c],
                device_id=y_nbr,
                device_id_type=pl.DeviceIdType.MESH,
            )

        def z_rdma(c):
            s = slot2(c)
            return pltpu.make_async_remote_copy(
                src_ref=zsend.at[pl.ds(s * MZ, MZ), :],
                dst_ref=zrecv.at[pl.ds(c * MZ, MZ), :],
                send_sem=zsend_sems.at[s],
                recv_sem=zrecv_sems.at[c],
                device_id=z_nbr,
                device_id_type=pl.DeviceIdType.MESH,
            )

        dy_dma(0).start()

        barrier = pltpu.get_barrier_semaphore()
        for nbr in (y_nbr, z_nbr):
            pl.semaphore_signal(
                barrier, inc=1, device_id=nbr,
                device_id_type=pl.DeviceIdType.MESH,
            )
        pl.semaphore_wait(barrier, 2)

        for i in range(NR):
            if i + 1 < NR:
                dy_dma(i + 1).start()
            dy_dma(i).wait()
            dy_bf[pl.ds(i * CR, CR), :] = stage[pl.ds((i % 2) * CN, CR), :].astype(BF)
            if i == NR - 2:
                w_dma(0).start()
            if i == NR - 1:
                w_dma(1).start()

        def loop_body(c, carry):
            s = lax.rem(c, 2)

            @pl.when(c < NC)
            def _compute():
                w_dma(c).wait()
                w_bf[pl.ds(s * CN, CN), :] = stage[pl.ds(s * CN, CN), :].astype(BF)

                @pl.when(c + 2 < NC)
                def _():
                    w_dma(c + 2).start()

                @pl.when(c >= 2)
                def _():
                    oy_dma(c - 2).wait()
                    y_rdma(c - 2).wait_send()

                p = lax.dot_general(
                    dy_bf[...], w_bf[pl.ds(s * CN, CN), :],
                    dimension_numbers=(((1,), (1,)), ((), ())),
                    preferred_element_type=F32,
                )
                p_buf[pl.ds(s * MZ, MZ), :] = p
                ysend[pl.ds(s * MZ, MZ), :] = p.astype(BF)
                y_rdma(c).start()

            @pl.when((c >= 1) & (c <= NC))
            def _reduce():
                k = c - 1
                sk = lax.rem(k, 2)
                y_rdma(k).wait_recv()
                p_buf[pl.ds(sk * MZ, MZ), :] = (
                    p_buf[pl.ds(sk * MZ, MZ), :]
                    + yrecv[pl.ds(k * MZ, MZ), :].astype(F32)
                )

                @pl.when(k >= 2)
                def _():
                    z_rdma(k - 2).wait_send()

                zsend[pl.ds(sk * MZ, MZ), :] = (
                    p_buf[pl.ds(sk * MZ, MZ), :].astype(BF)
                )
                z_rdma(k).start()
                oy_dma(k).start()

            @pl.when(c >= 2)
            def _gather():
                k = c - 2
                sk = lax.rem(k, 2)
                z_rdma(k).wait_recv()

                @pl.when(k >= 2)
                def _():
                    oz_dma(k - 2).wait()

                zf32[pl.ds(sk * MZ, MZ), :] = (
                    zrecv[pl.ds(k * MZ, MZ), :].astype(F32)
                )
                oz_dma(k).start()

            return carry

        lax.fori_loop(0, NC + 2, loop_body, 0)

        for k in (NC - 2, NC - 1):
            y_rdma(k).wait_send()
            z_rdma(k).wait_send()
            oy_dma(k).wait()
            oz_dma(k).wait()

    return pl.pallas_call(
        body,
        out_shape=jax.ShapeDtypeStruct((M, N), F32),
        in_specs=[
            pl.BlockSpec(memory_space=pl.ANY),
            pl.BlockSpec(memory_space=pl.ANY),
        ],
        out_specs=pl.BlockSpec(memory_space=pl.ANY),
        scratch_shapes=[
            pltpu.VMEM((2 * CN, K), F32),
            pltpu.VMEM((MZ, K), BF),
            pltpu.VMEM((2 * CN, K), BF),
            pltpu.VMEM((2 * MZ, CN), F32),
            pltpu.VMEM((2 * MZ, CN), BF),
            pltpu.VMEM((NC * MZ, CN), BF),
            pltpu.VMEM((2 * MZ, CN), BF),
            pltpu.VMEM((NC * MZ, CN), BF),
            pltpu.VMEM((2 * MZ, CN), F32),
            pltpu.SemaphoreType.DMA((2,)),
            pltpu.SemaphoreType.DMA((2,)),
            pltpu.SemaphoreType.DMA((NC,)),
            pltpu.SemaphoreType.DMA((2,)),
            pltpu.SemaphoreType.DMA((NC,)),
            pltpu.SemaphoreType.DMA((2,)),
            pltpu.SemaphoreType.DMA((2,)),
        ],
        compiler_params=pltpu.CompilerParams(
            collective_id=0,
            vmem_limit_bytes=63 * 1024 * 1024,
        ),
    )(dy, W)
